# baseline (speedup 1.0000x reference)
import functools

import jax
import jax.numpy as jnp
from jax import lax
from jax.experimental import pallas as pl
from jax.experimental.pallas import tpu as pltpu
from jax.experimental.pallas import tpu_sc as plsc

_VOCAB = 100000
_EMB = 50
_LANE = 128
_WIN = 5
_HID = 150
_TAGS = 45
_BATCH = 16384
_C = 128
_CP = _C // 2
_NB = _BATCH // _C
_NCHUNKS = _WIN * _NB


def _pack_pad_tc(t):
    n = t.shape[0]
    pb = 400
    nblk = n // 2 // pb

    def body(xt_ref, xb_ref, o_ref):
        z = jnp.zeros((pb, 64 - _EMB), jnp.float32)
        o_ref[...] = jnp.concatenate([xt_ref[...], z, xb_ref[...], z],
                                     axis=1)

    return pl.pallas_call(
        body,
        grid=(nblk,),
        in_specs=[pl.BlockSpec((pb, _EMB), lambda i: (i, 0)),
                  pl.BlockSpec((pb, _EMB), lambda i: (i + nblk, 0))],
        out_specs=pl.BlockSpec((pb, _LANE), lambda i: (i, 0)),
        out_shape=jax.ShapeDtypeStruct((n // 2, _LANE), jnp.float32),
    )(t, t)


def _remap(idx, half):
    return jnp.where(idx < half, 2 * idx, 2 * (idx - half) + 1)


def _gather_sum_sc(v_flat, pref_map, suff_map, e64, ep64, es64):
    info = plsc.get_sparse_core_info()
    nc, ns = info.num_cores, info.num_subcores
    nw = nc * ns
    per_w = _NCHUNKS // nw
    pairs = per_w // 2
    mesh = plsc.VectorSubcoreMesh(core_axis_name="c", subcore_axis_name="s")

    idx_t = pltpu.VMEM((_C,), jnp.int32)
    row_t = pltpu.VMEM((_C, 64), jnp.float32)
    out_t = pltpu.VMEM((_CP, _LANE), jnp.float32)

    @functools.partial(
        pl.kernel,
        mesh=mesh,
        compiler_params=pltpu.CompilerParams(use_tc_tiling_on_sc=False),
        out_type=jax.ShapeDtypeStruct((_WIN, _BATCH // 2, _LANE), jnp.float32),
        scratch_types=[
            idx_t, idx_t, idx_t, idx_t, idx_t, idx_t,
            row_t, row_t, row_t, row_t, row_t, row_t,
            out_t, out_t,
            pltpu.SemaphoreType.DMA, pltpu.SemaphoreType.DMA,
            pltpu.SemaphoreType.DMA, pltpu.SemaphoreType.DMA,
            pltpu.SemaphoreType.DMA, pltpu.SemaphoreType.DMA,
        ],
    )
    def gather_kernel(v_hbm, pm_hbm, sm_hbm, e_hbm, ep_hbm, es_hbm, out_hbm,
                      vi_a, vi_b, pi_a, pi_b, si_a, si_b,
                      be_a, bp_a, bs_a, be_b, bp_b, bs_b,
                      ob_a, ob_b,
                      sem_ia, sem_ib, sem_ra, sem_rb, sem_wa, sem_wb):
        wid = lax.axis_index("s") * nc + lax.axis_index("c")
        cbase = wid * per_w

        def load_idx(ci, vbuf):
            pltpu.sync_copy(v_hbm.at[pl.ds(ci * _C, _C)], vbuf)

        def start_maps(vbuf, pbuf, sbuf, sem):
            m1 = pltpu.async_copy(pm_hbm.at[vbuf], pbuf, sem)
            m2 = pltpu.async_copy(sm_hbm.at[vbuf], sbuf, sem)
            return m1, m2

        def start_rows(vbuf, pbuf, sbuf, be, bp, bs, sem):
            r1 = pltpu.async_copy(e_hbm.at[vbuf], be, sem)
            r2 = pltpu.async_copy(ep_hbm.at[pbuf], bp, sem)
            r3 = pltpu.async_copy(es_hbm.at[sbuf], bs, sem)
            return r1, r2, r3

        zeros16 = jnp.zeros((16,), jnp.float32)

        def zero_pad_cols(ob):
            def zrow(i, c2):
                ob[i, pl.ds(48, 16)] = zeros16
                ob[i, pl.ds(112, 16)] = zeros16
                return c2

            lax.fori_loop(0, _CP, zrow, 0)

        def add_chunk(be, bp, bs, ob):
            def row_body(i, c2):
                for half in (0, 1):
                    src = 2 * i + half
                    for o in (0, 16, 32, 34):
                        ob[i, pl.ds(64 * half + o, 16)] = (
                            be[src, pl.ds(o, 16)]
                            + bp[src, pl.ds(o, 16)]
                            + bs[src, pl.ds(o, 16)]
                        )
                return c2

            lax.fori_loop(0, _CP, row_body, 0)

        def start_wb(ci, ob, sem):
            w = ci // _NB
            p0 = (ci % _NB) * _CP
            return pltpu.async_copy(ob, out_hbm.at[w, pl.ds(p0, _CP)], sem)

        def wait_wb(ob, sem):
            pltpu.make_async_copy(ob, out_hbm.at[0, pl.ds(0, _CP)], sem).wait()

        zero_pad_cols(ob_a)
        zero_pad_cols(ob_b)

        load_idx(cbase, vi_a)
        m1, m2 = start_maps(vi_a, pi_a, si_a, sem_ia)
        m1.wait()
        m2.wait()

        def pair_body(j, carry):
            c0 = cbase + 2 * j
            c1 = c0 + 1
            c2 = lax.min(c0 + 2, _NCHUNKS - 1)

            r = start_rows(vi_a, pi_a, si_a, be_a, bp_a, bs_a, sem_ra)
            load_idx(c1, vi_b)
            mb = start_maps(vi_b, pi_b, si_b, sem_ib)

            @pl.when(j > 0)
            def _():
                wait_wb(ob_a, sem_wa)

            for d in r:
                d.wait()
            add_chunk(be_a, bp_a, bs_a, ob_a)
            start_wb(c0, ob_a, sem_wa)
            for d in mb:
                d.wait()

            r = start_rows(vi_b, pi_b, si_b, be_b, bp_b, bs_b, sem_rb)
            load_idx(c2, vi_a)
            ma = start_maps(vi_a, pi_a, si_a, sem_ia)

            @pl.when(j > 0)
            def _():
                wait_wb(ob_b, sem_wb)

            for d in r:
                d.wait()
            add_chunk(be_b, bp_b, bs_b, ob_b)
            start_wb(c1, ob_b, sem_wb)
            for d in ma:
                d.wait()
            return carry

        lax.fori_loop(0, pairs, pair_body, 0)
        wait_wb(ob_a, sem_wa)
        wait_wb(ob_b, sem_wb)

    return gather_kernel(v_flat, pref_map, suff_map, e64, ep64, es64)


def _mlp_tc(h, w0d, b0d, w1d, b1d):
    pblk = 512

    def body(h_ref, w0_ref, b0_ref, w1_ref, b1_ref, o_ref):
        acc = jnp.broadcast_to(b0_ref[...], (pblk, 2 * _HID))
        for w in range(_WIN):
            acc = acc + jnp.dot(h_ref[w], w0_ref[w],
                                preferred_element_type=jnp.float32)
        z = jnp.tanh(acc)
        logits = (
            jnp.dot(z, w1_ref[...], preferred_element_type=jnp.float32)
            + b1_ref[...]
        )
        halves = []
        for half in (0, 1):
            lg = logits[:, half * _TAGS:(half + 1) * _TAGS]
            m = jnp.max(lg, axis=1, keepdims=True)
            s = jnp.sum(jnp.exp(lg - m), axis=1, keepdims=True)
            halves.append(lg - (m + jnp.log(s)))
        o_ref[...] = jnp.concatenate(halves, axis=1)

    return pl.pallas_call(
        body,
        grid=(_BATCH // 2 // pblk,),
        in_specs=[
            pl.BlockSpec((_WIN, pblk, _LANE), lambda i: (0, i, 0)),
            pl.BlockSpec((_WIN, _LANE, 2 * _HID), lambda i: (0, 0, 0)),
            pl.BlockSpec((1, 2 * _HID), lambda i: (0, 0)),
            pl.BlockSpec((2 * _HID, 2 * _TAGS), lambda i: (0, 0)),
            pl.BlockSpec((1, 2 * _TAGS), lambda i: (0, 0)),
        ],
        out_specs=pl.BlockSpec((pblk, 2 * _TAGS), lambda i: (i, 0)),
        out_shape=jax.ShapeDtypeStruct((_BATCH // 2, 2 * _TAGS), jnp.float32),
    )(h, w0d, b0d, w1d, b1d)


def kernel(v, pref_map, suff_map, E, E_pref, E_suff, W0, b0, W1, b1):
    e64 = _pack_pad_tc(E).reshape(_VOCAB, 64)
    ep64 = _pack_pad_tc(E_pref).reshape(8000, 64)
    es64 = _pack_pad_tc(E_suff).reshape(8000, 64)
    v_flat = _remap(v.T.reshape(-1), _VOCAB // 2)
    vinv = jnp.arange(_VOCAB, dtype=jnp.int32)
    vinv = vinv // 2 + (vinv % 2) * (_VOCAB // 2)
    pm2 = _remap(pref_map[vinv], 4000)
    sm2 = _remap(suff_map[vinv], 4000)
    h = _gather_sum_sc(v_flat, pm2, sm2, e64, ep64, es64)

    w0p = jnp.pad(W0.reshape(_WIN, _EMB, _HID),
                  ((0, 0), (0, 64 - _EMB), (0, 0)))
    w0d = jnp.zeros((_WIN, _LANE, 2 * _HID), jnp.float32)
    w0d = w0d.at[:, :64, :_HID].set(w0p)
    w0d = w0d.at[:, 64:, _HID:].set(w0p)
    b0d = jnp.concatenate([b0, b0]).reshape(1, 2 * _HID)
    w1d = jnp.zeros((2 * _HID, 2 * _TAGS), jnp.float32)
    w1d = w1d.at[:_HID, :_TAGS].set(W1)
    w1d = w1d.at[_HID:, _TAGS:].set(W1)
    b1d = jnp.concatenate([b1, b1]).reshape(1, 2 * _TAGS)

    packed = _mlp_tc(h, w0d, b0d, w1d, b1d)
    return packed.reshape(_BATCH, _TAGS)


# speedup vs baseline: 1.3084x; 1.3084x over previous
import functools

import jax
import jax.numpy as jnp
from jax import lax
from jax.experimental import pallas as pl
from jax.experimental.pallas import tpu as pltpu
from jax.experimental.pallas import tpu_sc as plsc

_VOCAB = 100000
_EMB = 50
_EMBP = 64
_LANE = 128
_WIN = 5
_HID = 150
_TAGS = 45
_BATCH = 16384
_C = 128
_CP = _C // 2
_NB = _BATCH // _C
_NCHUNKS = _WIN * _NB


def _sc_mesh_info():
    info = plsc.get_sparse_core_info()
    return info.num_cores, info.num_subcores


_OUT_T = jax.ShapeDtypeStruct((_WIN, _BATCH // 2, _LANE), jnp.float32)


def _ps_gather_sc(v_flat, pref_map, suff_map, ep64, es64):
    nc, ns = _sc_mesh_info()
    per_w = _NCHUNKS // (nc * ns)
    pairs = per_w // 2
    mesh = plsc.VectorSubcoreMesh(core_axis_name="c", subcore_axis_name="s")

    idx_t = pltpu.VMEM((_C,), jnp.int32)
    row_t = pltpu.VMEM((_C, _EMBP), jnp.float32)
    out_t = pltpu.VMEM((_CP, _LANE), jnp.float32)

    @functools.partial(
        pl.kernel,
        mesh=mesh,
        compiler_params=pltpu.CompilerParams(use_tc_tiling_on_sc=False),
        out_type=_OUT_T,
        scratch_types=[
            idx_t, idx_t, idx_t, idx_t, idx_t, idx_t,
            row_t, row_t, row_t, row_t,
            out_t, out_t,
            pltpu.SemaphoreType.DMA, pltpu.SemaphoreType.DMA,
            pltpu.SemaphoreType.DMA, pltpu.SemaphoreType.DMA,
            pltpu.SemaphoreType.DMA, pltpu.SemaphoreType.DMA,
        ],
    )
    def k(v_hbm, pm_hbm, sm_hbm, ep_hbm, es_hbm, out_hbm,
          vi_a, vi_b, pi_a, pi_b, si_a, si_b,
          bp_a, bs_a, bp_b, bs_b, ob_a, ob_b,
          sem_ia, sem_ib, sem_ra, sem_rb, sem_wa, sem_wb):
        wid = lax.axis_index("s") * nc + lax.axis_index("c")
        cbase = wid * per_w

        def load_idx(ci, vbuf):
            pltpu.sync_copy(v_hbm.at[pl.ds(ci * _C, _C)], vbuf)

        def start_maps(vbuf, pbuf, sbuf, sem):
            return (pltpu.async_copy(pm_hbm.at[vbuf], pbuf, sem),
                    pltpu.async_copy(sm_hbm.at[vbuf], sbuf, sem))

        def start_rows(pbuf, sbuf, bp, bs, sem):
            return (pltpu.async_copy(ep_hbm.at[pbuf], bp, sem),
                    pltpu.async_copy(es_hbm.at[sbuf], bs, sem))

        zeros16 = jnp.zeros((16,), jnp.float32)

        def zero_pad_cols(ob):
            def zrow(i, c2):
                ob[i, pl.ds(48, 16)] = zeros16
                ob[i, pl.ds(112, 16)] = zeros16
                return c2

            lax.fori_loop(0, _CP, zrow, 0)

        def add_chunk(bp, bs, ob):
            def row_body(i, c2):
                for half in (0, 1):
                    src = 2 * i + half
                    for o in (0, 16, 32, 34):
                        ob[i, pl.ds(64 * half + o, 16)] = (
                            bp[src, pl.ds(o, 16)] + bs[src, pl.ds(o, 16)]
                        )
                return c2

            lax.fori_loop(0, _CP, row_body, 0)

        def start_wb(ci, ob, sem):
            w = ci // _NB
            p0 = (ci % _NB) * _CP
            return pltpu.async_copy(ob, out_hbm.at[w, pl.ds(p0, _CP)], sem)

        def wait_wb(ob, sem):
            pltpu.make_async_copy(ob, out_hbm.at[0, pl.ds(0, _CP)], sem).wait()

        zero_pad_cols(ob_a)
        zero_pad_cols(ob_b)
        load_idx(cbase, vi_a)
        m1, m2 = start_maps(vi_a, pi_a, si_a, sem_ia)
        m1.wait()
        m2.wait()

        def pair_body(j, carry):
            c0 = cbase + 2 * j
            c1 = c0 + 1
            c2 = lax.min(c0 + 2, _NCHUNKS - 1)

            r = start_rows(pi_a, si_a, bp_a, bs_a, sem_ra)
            load_idx(c1, vi_b)
            mb = start_maps(vi_b, pi_b, si_b, sem_ib)

            @pl.when(j > 0)
            def _():
                wait_wb(ob_a, sem_wa)

            for d in r:
                d.wait()
            add_chunk(bp_a, bs_a, ob_a)
            start_wb(c0, ob_a, sem_wa)
            for d in mb:
                d.wait()

            r = start_rows(pi_b, si_b, bp_b, bs_b, sem_rb)
            load_idx(c2, vi_a)
            ma = start_maps(vi_a, pi_a, si_a, sem_ia)

            @pl.when(j > 0)
            def _():
                wait_wb(ob_b, sem_wb)

            for d in r:
                d.wait()
            add_chunk(bp_b, bs_b, ob_b)
            start_wb(c1, ob_b, sem_wb)
            for d in ma:
                d.wait()
            return carry

        lax.fori_loop(0, pairs, pair_body, 0)
        wait_wb(ob_a, sem_wa)
        wait_wb(ob_b, sem_wb)

    return k(v_flat, pref_map, suff_map, ep64, es64)


def _e_gather_sc(v_flat, e64):
    nc, ns = _sc_mesh_info()
    per_w = _NCHUNKS // (nc * ns)
    pairs = per_w // 2
    mesh = plsc.VectorSubcoreMesh(core_axis_name="c", subcore_axis_name="s")

    idx_t = pltpu.VMEM((_C,), jnp.int32)
    row_t = pltpu.VMEM((_C, _EMBP), jnp.float32)
    out_t = pltpu.VMEM((_CP, _LANE), jnp.float32)

    @functools.partial(
        pl.kernel,
        mesh=mesh,
        compiler_params=pltpu.CompilerParams(use_tc_tiling_on_sc=False),
        out_type=_OUT_T,
        scratch_types=[
            idx_t, idx_t, row_t, row_t, out_t, out_t,
            pltpu.SemaphoreType.DMA, pltpu.SemaphoreType.DMA,
            pltpu.SemaphoreType.DMA, pltpu.SemaphoreType.DMA,
        ],
    )
    def k(v_hbm, e_hbm, out_hbm,
          vi_a, vi_b, be_a, be_b, ob_a, ob_b,
          sem_ra, sem_rb, sem_wa, sem_wb):
        wid = lax.axis_index("s") * nc + lax.axis_index("c")
        cbase = wid * per_w

        def load_idx(ci, vbuf):
            pltpu.sync_copy(v_hbm.at[pl.ds(ci * _C, _C)], vbuf)

        zeros16 = jnp.zeros((16,), jnp.float32)

        def zero_pad_cols(ob):
            def zrow(i, c2):
                ob[i, pl.ds(48, 16)] = zeros16
                ob[i, pl.ds(112, 16)] = zeros16
                return c2

            lax.fori_loop(0, _CP, zrow, 0)

        def repack_chunk(be, ob):
            def row_body(i, c2):
                for half in (0, 1):
                    src = 2 * i + half
                    for o in (0, 16, 32, 34):
                        ob[i, pl.ds(64 * half + o, 16)] = be[src, pl.ds(o, 16)]
                return c2

            lax.fori_loop(0, _CP, row_body, 0)

        def start_wb(ci, ob, sem):
            w = ci // _NB
            p0 = (ci % _NB) * _CP
            return pltpu.async_copy(ob, out_hbm.at[w, pl.ds(p0, _CP)], sem)

        def wait_wb(ob, sem):
            pltpu.make_async_copy(ob, out_hbm.at[0, pl.ds(0, _CP)], sem).wait()

        zero_pad_cols(ob_a)
        zero_pad_cols(ob_b)
        load_idx(cbase, vi_a)

        def pair_body(j, carry):
            c0 = cbase + 2 * j
            c1 = c0 + 1
            c2 = lax.min(c0 + 2, _NCHUNKS - 1)

            r = pltpu.async_copy(e_hbm.at[vi_a], be_a, sem_ra)
            load_idx(c1, vi_b)

            @pl.when(j > 0)
            def _():
                wait_wb(ob_a, sem_wa)

            r.wait()
            repack_chunk(be_a, ob_a)
            start_wb(c0, ob_a, sem_wa)

            r = pltpu.async_copy(e_hbm.at[vi_b], be_b, sem_rb)
            load_idx(c2, vi_a)

            @pl.when(j > 0)
            def _():
                wait_wb(ob_b, sem_wb)

            r.wait()
            repack_chunk(be_b, ob_b)
            start_wb(c1, ob_b, sem_wb)
            return carry

        lax.fori_loop(0, pairs, pair_body, 0)
        wait_wb(ob_a, sem_wa)
        wait_wb(ob_b, sem_wb)

    return k(v_flat, e64)


def _mlp_tc(h_ps, h_e, w0d, b0d, w1d, b1d):
    pblk = 512

    def body(hp_ref, he_ref, w0_ref, b0_ref, w1_ref, b1_ref, o_ref):
        acc = jnp.broadcast_to(b0_ref[...], (pblk, 2 * _HID))
        for w in range(_WIN):
            hw = hp_ref[w] + he_ref[w]
            acc = acc + jnp.dot(hw, w0_ref[w],
                                preferred_element_type=jnp.float32)
        z = jnp.tanh(acc)
        logits = (
            jnp.dot(z, w1_ref[...], preferred_element_type=jnp.float32)
            + b1_ref[...]
        )
        halves = []
        for half in (0, 1):
            lg = logits[:, half * _TAGS:(half + 1) * _TAGS]
            m = jnp.max(lg, axis=1, keepdims=True)
            s = jnp.sum(jnp.exp(lg - m), axis=1, keepdims=True)
            halves.append(lg - (m + jnp.log(s)))
        o_ref[...] = jnp.concatenate(halves, axis=1)

    h_spec = pl.BlockSpec((_WIN, pblk, _LANE), lambda i: (0, i, 0))
    return pl.pallas_call(
        body,
        grid=(_BATCH // 2 // pblk,),
        in_specs=[
            h_spec, h_spec,
            pl.BlockSpec((_WIN, _LANE, 2 * _HID), lambda i: (0, 0, 0)),
            pl.BlockSpec((1, 2 * _HID), lambda i: (0, 0)),
            pl.BlockSpec((2 * _HID, 2 * _TAGS), lambda i: (0, 0)),
            pl.BlockSpec((1, 2 * _TAGS), lambda i: (0, 0)),
        ],
        out_specs=pl.BlockSpec((pblk, 2 * _TAGS), lambda i: (i, 0)),
        out_shape=jax.ShapeDtypeStruct((_BATCH // 2, 2 * _TAGS), jnp.float32),
    )(h_ps, h_e, w0d, b0d, w1d, b1d)


def kernel(v, pref_map, suff_map, E, E_pref, E_suff, W0, b0, W1, b1):
    pad = ((0, 0), (0, _EMBP - _EMB))
    e64 = jnp.pad(E, pad)
    ep64 = jnp.pad(E_pref, pad)
    es64 = jnp.pad(E_suff, pad)
    v_flat = v.T.reshape(-1)

    h_ps = _ps_gather_sc(v_flat, pref_map, suff_map, ep64, es64)
    h_e = _e_gather_sc(v_flat, e64)

    w0p = jnp.pad(W0.reshape(_WIN, _EMB, _HID),
                  ((0, 0), (0, _EMBP - _EMB), (0, 0)))
    w0d = jnp.zeros((_WIN, _LANE, 2 * _HID), jnp.float32)
    w0d = w0d.at[:, :_EMBP, :_HID].set(w0p)
    w0d = w0d.at[:, _EMBP:, _HID:].set(w0p)
    b0d = jnp.concatenate([b0, b0]).reshape(1, 2 * _HID)
    w1d = jnp.zeros((2 * _HID, 2 * _TAGS), jnp.float32)
    w1d = w1d.at[:_HID, :_TAGS].set(W1)
    w1d = w1d.at[_HID:, _TAGS:].set(W1)
    b1d = jnp.concatenate([b1, b1]).reshape(1, 2 * _TAGS)

    packed = _mlp_tc(h_ps, h_e, w0d, b0d, w1d, b1d)
    return packed.reshape(_BATCH, _TAGS)


# speedup vs baseline: 1.5448x; 1.1807x over previous
import functools

import jax
import jax.numpy as jnp
from jax import lax
from jax.experimental import pallas as pl
from jax.experimental.pallas import tpu as pltpu
from jax.experimental.pallas import tpu_sc as plsc

_VOCAB = 100000
_EMB = 50
_EMBP = 64
_LANE = 128
_WIN = 5
_HID = 150
_TAGS = 45
_BATCH = 16384
_C = 128
_CP = _C // 2
_NB = _BATCH // _C
_NCHUNKS = _WIN * _NB


def _gather_sum_sc(vi_flat, ve_flat, pref_map, suff_map, e64, ep64, es64):
    info = plsc.get_sparse_core_info()
    nc, ns = info.num_cores, info.num_subcores
    per_w = _NCHUNKS // (nc * ns)
    pairs = per_w // 2
    mesh = plsc.VectorSubcoreMesh(core_axis_name="c", subcore_axis_name="s")

    idx_t = pltpu.VMEM((_C,), jnp.int32)
    row_t = pltpu.VMEM((_C, _EMBP), jnp.float32)
    out_t = pltpu.VMEM((_CP, _LANE), jnp.float32)

    @functools.partial(
        pl.kernel,
        mesh=mesh,
        compiler_params=pltpu.CompilerParams(use_tc_tiling_on_sc=False),
        out_type=jax.ShapeDtypeStruct((_WIN, _BATCH // 2, _LANE), jnp.float32),
        scratch_types=[
            idx_t, idx_t, idx_t, idx_t, idx_t, idx_t, idx_t, idx_t,
            row_t, row_t, row_t, row_t, row_t, row_t,
            out_t, out_t,
            pltpu.SemaphoreType.DMA, pltpu.SemaphoreType.DMA,
            pltpu.SemaphoreType.DMA, pltpu.SemaphoreType.DMA,
            pltpu.SemaphoreType.DMA, pltpu.SemaphoreType.DMA,
        ],
    )
    def gather_kernel(vi_hbm, ve_hbm, pm_hbm, sm_hbm, e_hbm, ep_hbm, es_hbm,
                      out_hbm,
                      vi_a, vi_b, ve_a, ve_b, pi_a, pi_b, si_a, si_b,
                      be_a, bp_a, bs_a, be_b, bp_b, bs_b,
                      ob_a, ob_b,
                      sem_ia, sem_ib, sem_ra, sem_rb, sem_wa, sem_wb):
        wid = lax.axis_index("s") * nc + lax.axis_index("c")
        cbase = wid * per_w

        def load_idx(ci, vbuf, vebuf):
            pltpu.sync_copy(vi_hbm.at[pl.ds(ci * _C, _C)], vbuf)
            pltpu.sync_copy(ve_hbm.at[pl.ds(ci * _C, _C)], vebuf)

        def start_maps(vbuf, pbuf, sbuf, sem):
            return (pltpu.async_copy(pm_hbm.at[vbuf], pbuf, sem),
                    pltpu.async_copy(sm_hbm.at[vbuf], sbuf, sem))

        def start_rows(vebuf, pbuf, sbuf, be, bp, bs, sem):
            return (pltpu.async_copy(e_hbm.at[vebuf], be, sem),
                    pltpu.async_copy(ep_hbm.at[pbuf], bp, sem),
                    pltpu.async_copy(es_hbm.at[sbuf], bs, sem))

        zeros16 = jnp.zeros((16,), jnp.float32)

        def zero_pad_cols(ob):
            def zrow(i, c2):
                ob[i, pl.ds(48, 16)] = zeros16
                ob[i, pl.ds(112, 16)] = zeros16
                return c2

            lax.fori_loop(0, _CP, zrow, 0)

        def add_chunk(be, bp, bs, ob):
            def row_body(i, c2):
                for half in (0, 1):
                    src = 2 * i + half
                    for o in (0, 16, 32, 34):
                        ob[i, pl.ds(64 * half + o, 16)] = (
                            be[src, pl.ds(o, 16)]
                            + bp[src, pl.ds(o, 16)]
                            + bs[src, pl.ds(o, 16)]
                        )
                return c2

            lax.fori_loop(0, _CP, row_body, 0)

        def start_wb(ci, ob, sem):
            w = ci // _NB
            p0 = (ci % _NB) * _CP
            return pltpu.async_copy(ob, out_hbm.at[w, pl.ds(p0, _CP)], sem)

        def wait_wb(ob, sem):
            pltpu.make_async_copy(ob, out_hbm.at[0, pl.ds(0, _CP)], sem).wait()

        zero_pad_cols(ob_a)
        zero_pad_cols(ob_b)
        load_idx(cbase, vi_a, ve_a)
        m1, m2 = start_maps(vi_a, pi_a, si_a, sem_ia)
        m1.wait()
        m2.wait()

        def pair_body(j, carry):
            c0 = cbase + 2 * j
            c1 = c0 + 1
            c2 = lax.min(c0 + 2, _NCHUNKS - 1)

            r = start_rows(ve_a, pi_a, si_a, be_a, bp_a, bs_a, sem_ra)
            load_idx(c1, vi_b, ve_b)
            mb = start_maps(vi_b, pi_b, si_b, sem_ib)

            @pl.when(j > 0)
            def _():
                wait_wb(ob_a, sem_wa)

            for d in r:
                d.wait()
            add_chunk(be_a, bp_a, bs_a, ob_a)
            start_wb(c0, ob_a, sem_wa)
            for d in mb:
                d.wait()

            r = start_rows(ve_b, pi_b, si_b, be_b, bp_b, bs_b, sem_rb)
            load_idx(c2, vi_a, ve_a)
            ma = start_maps(vi_a, pi_a, si_a, sem_ia)

            @pl.when(j > 0)
            def _():
                wait_wb(ob_b, sem_wb)

            for d in r:
                d.wait()
            add_chunk(be_b, bp_b, bs_b, ob_b)
            start_wb(c1, ob_b, sem_wb)
            for d in ma:
                d.wait()
            return carry

        lax.fori_loop(0, pairs, pair_body, 0)
        wait_wb(ob_a, sem_wa)
        wait_wb(ob_b, sem_wb)

    return gather_kernel(vi_flat, ve_flat, pref_map, suff_map,
                         e64, ep64, es64)


def _mlp_tc(h, w0d, b0d, w1d, b1d):
    pblk = 512

    def body(h_ref, w0_ref, b0_ref, w1_ref, b1_ref, o_ref):
        acc = jnp.broadcast_to(b0_ref[...], (pblk, 2 * _HID))
        for w in range(_WIN):
            acc = acc + jnp.dot(h_ref[w], w0_ref[w],
                                preferred_element_type=jnp.float32)
        z = jnp.tanh(acc)
        logits = (
            jnp.dot(z, w1_ref[...], preferred_element_type=jnp.float32)
            + b1_ref[...]
        )
        halves = []
        for half in (0, 1):
            lg = logits[:, half * _TAGS:(half + 1) * _TAGS]
            m = jnp.max(lg, axis=1, keepdims=True)
            s = jnp.sum(jnp.exp(lg - m), axis=1, keepdims=True)
            halves.append(lg - (m + jnp.log(s)))
        o_ref[...] = jnp.concatenate(halves, axis=1)

    return pl.pallas_call(
        body,
        grid=(_BATCH // 2 // pblk,),
        in_specs=[
            pl.BlockSpec((_WIN, pblk, _LANE), lambda i: (0, i, 0)),
            pl.BlockSpec((_WIN, _LANE, 2 * _HID), lambda i: (0, 0, 0)),
            pl.BlockSpec((1, 2 * _HID), lambda i: (0, 0)),
            pl.BlockSpec((2 * _HID, 2 * _TAGS), lambda i: (0, 0)),
            pl.BlockSpec((1, 2 * _TAGS), lambda i: (0, 0)),
        ],
        out_specs=pl.BlockSpec((pblk, 2 * _TAGS), lambda i: (i, 0)),
        out_shape=jax.ShapeDtypeStruct((_BATCH // 2, 2 * _TAGS), jnp.float32),
    )(h, w0d, b0d, w1d, b1d)


def kernel(v, pref_map, suff_map, E, E_pref, E_suff, W0, b0, W1, b1):
    half_v = _VOCAB // 2
    ar = jnp.arange(_EMB)
    s1 = jnp.zeros((_EMB, _LANE), jnp.float32).at[ar, ar].set(1.0)
    s2 = jnp.zeros((_EMB, _LANE), jnp.float32).at[ar, ar + _EMBP].set(1.0)
    e_pack = (jnp.dot(E[:half_v], s1, preferred_element_type=jnp.float32)
              + jnp.dot(E[half_v:], s2, preferred_element_type=jnp.float32))
    e64 = e_pack.reshape(_VOCAB, _EMBP)

    pad = ((0, 0), (0, _EMBP - _EMB))
    ep64 = jnp.pad(E_pref, pad)
    es64 = jnp.pad(E_suff, pad)

    vt = v.T.reshape(-1)
    ve_flat = jnp.where(vt < half_v, 2 * vt, 2 * (vt - half_v) + 1)
    h = _gather_sum_sc(vt, ve_flat, pref_map, suff_map, e64, ep64, es64)

    w0p = jnp.pad(W0.reshape(_WIN, _EMB, _HID),
                  ((0, 0), (0, _EMBP - _EMB), (0, 0)))
    w0d = jnp.zeros((_WIN, _LANE, 2 * _HID), jnp.float32)
    w0d = w0d.at[:, :_EMBP, :_HID].set(w0p)
    w0d = w0d.at[:, _EMBP:, _HID:].set(w0p)
    b0d = jnp.concatenate([b0, b0]).reshape(1, 2 * _HID)
    w1d = jnp.zeros((2 * _HID, 2 * _TAGS), jnp.float32)
    w1d = w1d.at[:_HID, :_TAGS].set(W1)
    w1d = w1d.at[_HID:, _TAGS:].set(W1)
    b1d = jnp.concatenate([b1, b1]).reshape(1, 2 * _TAGS)

    packed = _mlp_tc(h, w0d, b0d, w1d, b1d)
    return packed.reshape(_BATCH, _TAGS)


# speedup vs baseline: 1.6608x; 1.0751x over previous
import functools

import jax
import jax.numpy as jnp
from jax import lax
from jax.experimental import pallas as pl
from jax.experimental.pallas import tpu as pltpu
from jax.experimental.pallas import tpu_sc as plsc

_VOCAB = 100000
_EMB = 50
_EMBP = 64
_LANE = 128
_WIN = 5
_HID = 150
_TAGS = 45
_BATCH = 16384
_C = 128
_CP = _C // 2
_NB = _BATCH // _C
_NCHUNKS = _WIN * _NB


def _gather_sum_sc(vi_flat, ve_flat, pref_map, suff_map, e64, ep64, es64,
                   batch):
    nb = batch // _C
    nchunks = _WIN * nb
    info = plsc.get_sparse_core_info()
    nc, ns = info.num_cores, info.num_subcores
    per_w = nchunks // (nc * ns)
    pairs = per_w // 2
    mesh = plsc.VectorSubcoreMesh(core_axis_name="c", subcore_axis_name="s")

    idx_t = pltpu.VMEM((_C,), jnp.int32)
    row_t = pltpu.VMEM((_C, _EMBP), jnp.float32)
    out_t = pltpu.VMEM((_CP, _LANE), jnp.float32)

    @functools.partial(
        pl.kernel,
        mesh=mesh,
        compiler_params=pltpu.CompilerParams(use_tc_tiling_on_sc=False),
        out_type=jax.ShapeDtypeStruct((_WIN, batch // 2, _LANE), jnp.float32),
        scratch_types=[
            idx_t, idx_t, idx_t, idx_t, idx_t, idx_t, idx_t, idx_t,
            row_t, row_t, row_t, row_t, row_t, row_t,
            out_t, out_t,
            pltpu.SemaphoreType.DMA, pltpu.SemaphoreType.DMA,
            pltpu.SemaphoreType.DMA, pltpu.SemaphoreType.DMA,
            pltpu.SemaphoreType.DMA, pltpu.SemaphoreType.DMA,
        ],
    )
    def gather_kernel(vi_hbm, ve_hbm, pm_hbm, sm_hbm, e_hbm, ep_hbm, es_hbm,
                      out_hbm,
                      vi_a, vi_b, ve_a, ve_b, pi_a, pi_b, si_a, si_b,
                      be_a, bp_a, bs_a, be_b, bp_b, bs_b,
                      ob_a, ob_b,
                      sem_ia, sem_ib, sem_ra, sem_rb, sem_wa, sem_wb):
        wid = lax.axis_index("s") * nc + lax.axis_index("c")
        cbase = wid * per_w

        def load_idx(ci, vbuf, vebuf):
            pltpu.sync_copy(vi_hbm.at[pl.ds(ci * _C, _C)], vbuf)
            pltpu.sync_copy(ve_hbm.at[pl.ds(ci * _C, _C)], vebuf)

        def start_maps(vbuf, pbuf, sbuf, sem):
            return (pltpu.async_copy(pm_hbm.at[vbuf], pbuf, sem),
                    pltpu.async_copy(sm_hbm.at[vbuf], sbuf, sem))

        def start_rows(vebuf, pbuf, sbuf, be, bp, bs, sem):
            return (pltpu.async_copy(e_hbm.at[vebuf], be, sem),
                    pltpu.async_copy(ep_hbm.at[pbuf], bp, sem),
                    pltpu.async_copy(es_hbm.at[sbuf], bs, sem))

        zeros16 = jnp.zeros((16,), jnp.float32)

        def zero_pad_cols(ob):
            def zrow(i, c2):
                ob[i, pl.ds(48, 16)] = zeros16
                ob[i, pl.ds(112, 16)] = zeros16
                return c2

            lax.fori_loop(0, _CP, zrow, 0)

        def add_chunk(be, bp, bs, ob):
            def row_body(i, c2):
                for half in (0, 1):
                    src = 2 * i + half
                    for o in (0, 16, 32, 34):
                        ob[i, pl.ds(64 * half + o, 16)] = (
                            be[src, pl.ds(o, 16)]
                            + bp[src, pl.ds(o, 16)]
                            + bs[src, pl.ds(o, 16)]
                        )
                return c2

            lax.fori_loop(0, _CP, row_body, 0)

        def start_wb(ci, ob, sem):
            w = ci // nb
            p0 = (ci % nb) * _CP
            return pltpu.async_copy(ob, out_hbm.at[w, pl.ds(p0, _CP)], sem)

        def wait_wb(ob, sem):
            pltpu.make_async_copy(ob, out_hbm.at[0, pl.ds(0, _CP)], sem).wait()

        zero_pad_cols(ob_a)
        zero_pad_cols(ob_b)
        load_idx(cbase, vi_a, ve_a)
        m1, m2 = start_maps(vi_a, pi_a, si_a, sem_ia)
        m1.wait()
        m2.wait()

        def pair_body(j, carry):
            c0 = cbase + 2 * j
            c1 = c0 + 1
            c2 = lax.min(c0 + 2, nchunks - 1)

            r = start_rows(ve_a, pi_a, si_a, be_a, bp_a, bs_a, sem_ra)
            load_idx(c1, vi_b, ve_b)
            mb = start_maps(vi_b, pi_b, si_b, sem_ib)

            @pl.when(j > 0)
            def _():
                wait_wb(ob_a, sem_wa)

            for d in r:
                d.wait()
            add_chunk(be_a, bp_a, bs_a, ob_a)
            start_wb(c0, ob_a, sem_wa)
            for d in mb:
                d.wait()

            r = start_rows(ve_b, pi_b, si_b, be_b, bp_b, bs_b, sem_rb)
            load_idx(c2, vi_a, ve_a)
            ma = start_maps(vi_a, pi_a, si_a, sem_ia)

            @pl.when(j > 0)
            def _():
                wait_wb(ob_b, sem_wb)

            for d in r:
                d.wait()
            add_chunk(be_b, bp_b, bs_b, ob_b)
            start_wb(c1, ob_b, sem_wb)
            for d in ma:
                d.wait()
            return carry

        lax.fori_loop(0, pairs, pair_body, 0)
        wait_wb(ob_a, sem_wa)
        wait_wb(ob_b, sem_wb)

    return gather_kernel(vi_flat, ve_flat, pref_map, suff_map,
                         e64, ep64, es64)


def _mlp_tc(h, w0d, b0d, w1d, b1d, batch):
    pblk = 512

    def body(h_ref, w0_ref, b0_ref, w1_ref, b1_ref, o_ref):
        acc = jnp.broadcast_to(b0_ref[...], (pblk, 2 * _HID))
        for w in range(_WIN):
            acc = acc + jnp.dot(h_ref[w], w0_ref[w],
                                preferred_element_type=jnp.float32)
        z = jnp.tanh(acc)
        logits = (
            jnp.dot(z, w1_ref[...], preferred_element_type=jnp.float32)
            + b1_ref[...]
        )
        halves = []
        for half in (0, 1):
            lg = logits[:, half * _TAGS:(half + 1) * _TAGS]
            m = jnp.max(lg, axis=1, keepdims=True)
            s = jnp.sum(jnp.exp(lg - m), axis=1, keepdims=True)
            halves.append(lg - (m + jnp.log(s)))
        o_ref[...] = jnp.concatenate(halves, axis=1)

    return pl.pallas_call(
        body,
        grid=(batch // 2 // pblk,),
        in_specs=[
            pl.BlockSpec((_WIN, pblk, _LANE), lambda i: (0, i, 0)),
            pl.BlockSpec((_WIN, _LANE, 2 * _HID), lambda i: (0, 0, 0)),
            pl.BlockSpec((1, 2 * _HID), lambda i: (0, 0)),
            pl.BlockSpec((2 * _HID, 2 * _TAGS), lambda i: (0, 0)),
            pl.BlockSpec((1, 2 * _TAGS), lambda i: (0, 0)),
        ],
        out_specs=pl.BlockSpec((pblk, 2 * _TAGS), lambda i: (i, 0)),
        out_shape=jax.ShapeDtypeStruct((batch // 2, 2 * _TAGS), jnp.float32),
    )(h, w0d, b0d, w1d, b1d)


def kernel(v, pref_map, suff_map, E, E_pref, E_suff, W0, b0, W1, b1):
    half_v = _VOCAB // 2
    ar = jnp.arange(_EMB)
    s1 = jnp.zeros((_EMB, _LANE), jnp.float32).at[ar, ar].set(1.0)
    s2 = jnp.zeros((_EMB, _LANE), jnp.float32).at[ar, ar + _EMBP].set(1.0)
    e_pack = (jnp.dot(E[:half_v], s1, preferred_element_type=jnp.float32)
              + jnp.dot(E[half_v:], s2, preferred_element_type=jnp.float32))
    e64 = e_pack.reshape(_VOCAB, _EMBP)

    pad = ((0, 0), (0, _EMBP - _EMB))
    ep64 = jnp.pad(E_pref, pad)
    es64 = jnp.pad(E_suff, pad)

    w0p = jnp.pad(W0.reshape(_WIN, _EMB, _HID),
                  ((0, 0), (0, _EMBP - _EMB), (0, 0)))
    w0d = jnp.zeros((_WIN, _LANE, 2 * _HID), jnp.float32)
    w0d = w0d.at[:, :_EMBP, :_HID].set(w0p)
    w0d = w0d.at[:, _EMBP:, _HID:].set(w0p)
    b0d = jnp.concatenate([b0, b0]).reshape(1, 2 * _HID)
    w1d = jnp.zeros((2 * _HID, 2 * _TAGS), jnp.float32)
    w1d = w1d.at[:_HID, :_TAGS].set(W1)
    w1d = w1d.at[_HID:, _TAGS:].set(W1)
    b1d = jnp.concatenate([b1, b1]).reshape(1, 2 * _TAGS)

    hb = _BATCH // 2
    vt2 = v.T
    outs = []
    hs = []
    for p in range(2):
        vth = vt2[:, p * hb:(p + 1) * hb].reshape(-1)
        veh = jnp.where(vth < half_v, 2 * vth, 2 * (vth - half_v) + 1)
        hs.append(_gather_sum_sc(vth, veh, pref_map, suff_map,
                                 e64, ep64, es64, hb))
    for p in range(2):
        packed = _mlp_tc(hs[p], w0d, b0d, w1d, b1d, hb)
        outs.append(packed.reshape(hb, _TAGS))
    return jnp.concatenate(outs, axis=0)
